# baseline (speedup 1.0000x reference)
import functools

import numpy as np
import jax
import jax.numpy as jnp
from jax import lax
from jax.experimental import pallas as pl
from jax.experimental.pallas import tpu as pltpu
from jax.experimental.pallas import tpu_sc as plsc

_N = 10000
_E = 320000
_D = 128
_G = 64
_NCLS = 16

_NC, _NS = 2, 16
_NP = 10112
_RPT = _NP // _NS
_CHUNK = 128
_CH0 = 160
_CH1 = 0
_E0 = _NS * _CH0 * _CHUNK
_EPAD = _NS * (_CH0 + _CH1) * _CHUNK

_BN = 1000
_NBLK = _N // _BN

_BNSCALE = float(1.0 / np.sqrt(1.0 + 1e-5))


def _dot(a, b):
    return lax.dot_general(a, b, (((1,), (0,)), ((), ())),
                           precision=lax.Precision.HIGHEST,
                           preferred_element_type=jnp.float32)


@functools.cache
def _get_sc_agg():
    mesh = plsc.VectorSubcoreMesh(core_axis_name="c", subcore_axis_name="s",
                                  num_cores=_NC, num_subcores=_NS)

    @functools.partial(
        pl.kernel,
        mesh=mesh,
        out_type=jax.ShapeDtypeStruct((_NC, _NP, _D), jnp.float32),
        scratch_types=[
            pltpu.VMEM((80, _CHUNK), jnp.int32),
            pltpu.VMEM((80, _CHUNK), jnp.int32),
            pltpu.VMEM((_CHUNK, _D), jnp.float32),
            pltpu.VMEM_SHARED((_NP, _D), jnp.float32),
            pltpu.SemaphoreType.DMA,
        ],
    )
    def _sc_agg(x_hbm, src0_hbm, dst0_hbm, src1_hbm, dst1_hbm, zeros_hbm,
                out_hbm, src_v, dst_v, rows_v, agg_sh, sem):
        c = lax.axis_index("c")
        s = lax.axis_index("s")
        pltpu.sync_copy(zeros_hbm.at[pl.ds(s * _RPT, _RPT)],
                        agg_sh.at[pl.ds(s * _RPT, _RPT)])

        def run(src_hbm_c, dst_hbm_c, ch):
            plsc.subcore_barrier()

            def body(j, carry):
                pltpu.async_copy(x_hbm.at[src_v.at[j]], rows_v, sem).wait()
                pltpu.sync_copy(rows_v, agg_sh.at[dst_v.at[j]], add=True)
                return carry

            for base in range(0, ch, 80):
                n = min(80, ch - base)
                pltpu.sync_copy(src_hbm_c.at[s, pl.ds(base, n)],
                                src_v.at[pl.ds(0, n)])
                pltpu.sync_copy(dst_hbm_c.at[s, pl.ds(base, n)],
                                dst_v.at[pl.ds(0, n)])
                lax.fori_loop(0, n, body, 0)

        @pl.when(c == 0)
        def _():
            run(src0_hbm, dst0_hbm, _CH0)

        if _CH1:
            @pl.when(c == 1)
            def _():
                run(src1_hbm, dst1_hbm, _CH1)

        plsc.subcore_barrier()
        pltpu.sync_copy(agg_sh.at[pl.ds(s * _RPT, _RPT)],
                        out_hbm.at[c, pl.ds(s * _RPT, _RPT)])

    return _sc_agg


def _gin_mlp(x, agg, eps, wa, ba, wb, bb, g, be):
    h = (1.0 + eps) * x + agg
    h = jnp.maximum(_dot(h, wa) + ba, 0.0)
    h = jnp.maximum(_dot(h, wb) + bb, 0.0)
    return h * (g * _BNSCALE) + be


def _mlp_body(eps_ref, x_ref, agg_ref, wa_ref, ba_ref, wb_ref, bb_ref,
              g_ref, be_ref, out_ref):
    out_ref[...] = _gin_mlp(x_ref[...], agg_ref[0] + agg_ref[1], eps_ref[0],
                            wa_ref[...], ba_ref[...], wb_ref[...], bb_ref[...],
                            g_ref[...], be_ref[...])


_w_spec = pl.BlockSpec((_D, _D), lambda i: (0, 0))
_v_spec = pl.BlockSpec((1, _D), lambda i: (0, 0))

_mlp_call = pl.pallas_call(
    _mlp_body,
    grid=(_NBLK,),
    in_specs=[
        pl.BlockSpec(memory_space=pltpu.SMEM),
        pl.BlockSpec((_BN, _D), lambda i: (i, 0)),
        pl.BlockSpec((_NC, _BN, _D), lambda i: (0, i, 0)),
        _w_spec, _v_spec, _w_spec, _v_spec, _v_spec, _v_spec,
    ],
    out_specs=pl.BlockSpec((_BN, _D), lambda i: (i, 0)),
    out_shape=jax.ShapeDtypeStruct((_N, _D), jnp.float32),
)


def _final_body(eps_ref, x2_ref, agg_ref, wa_ref, ba_ref, wb_ref, bb_ref,
                g_ref, be_ref, x1_ref, batch_ref, emb_ref,
                w1_ref, b1_ref, w2_ref, b2_ref, w4_ref, b4_ref,
                out_ref, pooled_ref, counts_ref):
    i = pl.program_id(0)
    x2 = x2_ref[...]
    x3 = _gin_mlp(x2, agg_ref[0] + agg_ref[1], eps_ref[0],
                  wa_ref[...], ba_ref[...], wb_ref[...], bb_ref[...],
                  g_ref[...], be_ref[...])
    ids = batch_ref[0]
    gidx = lax.broadcasted_iota(jnp.int32, (_G, _BN), 0)
    oh = (gidx == ids).astype(jnp.float32)
    cat = jnp.concatenate([x1_ref[...], x2, x3], axis=1)
    contrib = _dot(oh, cat)
    cnt = jnp.sum(oh, axis=1, keepdims=True)

    @pl.when(i == 0)
    def _():
        pooled_ref[...] = jnp.zeros_like(pooled_ref)
        counts_ref[...] = jnp.zeros_like(counts_ref)

    pooled_ref[...] += contrib
    counts_ref[...] += jnp.broadcast_to(cnt, counts_ref.shape)

    @pl.when(i == _NBLK - 1)
    def _():
        c = jnp.maximum(counts_ref[...], 1.0)
        pooled = pooled_ref[...] / jnp.concatenate([c, c, c], axis=1)
        z = jnp.concatenate([pooled, emb_ref[...]], axis=1)
        z = jnp.maximum(_dot(z, w1_ref[...]) + b1_ref[...], 0.0)
        z = jnp.maximum(_dot(z, w2_ref[...]) + b2_ref[...], 0.0)
        z = _dot(z, w4_ref[...]) + b4_ref[...]
        m = jnp.max(z, axis=1, keepdims=True)
        lse = jnp.log(jnp.sum(jnp.exp(z - m), axis=1, keepdims=True)) + m
        out_ref[...] = z - lse


_final_call = pl.pallas_call(
    _final_body,
    grid=(_NBLK,),
    in_specs=[
        pl.BlockSpec(memory_space=pltpu.SMEM),
        pl.BlockSpec((_BN, _D), lambda i: (i, 0)),
        pl.BlockSpec((_NC, _BN, _D), lambda i: (0, i, 0)),
        _w_spec, _v_spec, _w_spec, _v_spec, _v_spec, _v_spec,
        pl.BlockSpec((_BN, _D), lambda i: (i, 0)),
        pl.BlockSpec((1, 1, _BN), lambda i: (i, 0, 0)),
        pl.BlockSpec((_G, _D), lambda i: (0, 0)),
        pl.BlockSpec((3 * _D + _D, 2 * _D), lambda i: (0, 0)),
        pl.BlockSpec((1, 2 * _D), lambda i: (0, 0)),
        pl.BlockSpec((2 * _D, _D), lambda i: (0, 0)),
        _v_spec,
        pl.BlockSpec((_D, _NCLS), lambda i: (0, 0)),
        pl.BlockSpec((1, _NCLS), lambda i: (0, 0)),
    ],
    out_specs=pl.BlockSpec((_G, _NCLS), lambda i: (0, 0)),
    out_shape=jax.ShapeDtypeStruct((_G, _NCLS), jnp.float32),
    scratch_shapes=[
        pltpu.VMEM((_G, 3 * _D), jnp.float32),
        pltpu.VMEM((_G, _D), jnp.float32),
    ],
)


def kernel(x, edge_index, batch, emb,
           eps0, W0a, b0a, W0b, b0b, g0, be0,
           eps1, W1a, b1a, W1b, b1b, g1, be1,
           eps2, W2a, b2a, W2b, b2b, g2, be2,
           W_lin1, b_lin1, W_lin2, b_lin2, W_lin4, b_lin4):
    src = edge_index[0]
    dst = edge_index[1]
    pad = _EPAD - _E
    src_p = jnp.concatenate([src, jnp.zeros((pad,), jnp.int32)])
    src0 = src_p[:_E0].reshape(_NS, _CH0, _CHUNK)
    dst_p = jnp.concatenate([dst, jnp.full((pad,), _N, jnp.int32)])
    dst0 = dst_p[:_E0].reshape(_NS, _CH0, _CHUNK)
    if _CH1:
        src1 = src_p[_E0:].reshape(_NS, _CH1, _CHUNK)
        dst1 = dst_p[_E0:].reshape(_NS, _CH1, _CHUNK)
    else:
        src1, dst1 = src0, dst0
    zeros = jnp.zeros((_NP, _D), jnp.float32)
    batch_r = batch.reshape(_NBLK, 1, _BN)

    def row(v):
        return v.reshape(1, -1)

    sc_agg = _get_sc_agg()
    agg1 = sc_agg(x, src0, dst0, src1, dst1, zeros)
    x1 = _mlp_call(eps0.reshape(1), x, agg1, W0a, row(b0a), W0b, row(b0b),
                   row(g0), row(be0))
    agg2 = sc_agg(x1, src0, dst0, src1, dst1, zeros)
    x2 = _mlp_call(eps1.reshape(1), x1, agg2, W1a, row(b1a), W1b, row(b1b),
                   row(g1), row(be1))
    agg3 = sc_agg(x2, src0, dst0, src1, dst1, zeros)
    out = _final_call(eps2.reshape(1), x2, agg3, W2a, row(b2a), W2b, row(b2b),
                      row(g2), row(be2), x1, batch_r, emb,
                      W_lin1, row(b_lin1), W_lin2, row(b_lin2),
                      W_lin4, row(b_lin4))
    return out


# speedup vs baseline: 1.8318x; 1.8318x over previous
import functools

import numpy as np
import jax
import jax.numpy as jnp
from jax import lax
from jax.experimental import pallas as pl
from jax.experimental.pallas import tpu as pltpu
from jax.experimental.pallas import tpu_sc as plsc

_N = 10000
_E = 320000
_D = 128
_G = 64
_NCLS = 16

_NC, _NS = 2, 16
_NP = 10112
_RPT = _NP // _NS
_CHUNK = 128
_CH = 79
_EPAD = _NC * _NS * _CH * _CHUNK

_BN = 1000
_NBLK = _N // _BN

_BNSCALE = float(1.0 / np.sqrt(1.0 + 1e-5))


def _dot(a, b):
    return lax.dot_general(a, b, (((1,), (0,)), ((), ())),
                           precision=lax.Precision.HIGHEST,
                           preferred_element_type=jnp.float32)


@functools.cache
def _get_sc_agg():
    mesh = plsc.VectorSubcoreMesh(core_axis_name="c", subcore_axis_name="s",
                                  num_cores=_NC, num_subcores=_NS)

    @functools.partial(
        pl.kernel,
        mesh=mesh,
        out_type=jax.ShapeDtypeStruct((_NC, _NP, _D), jnp.float32),
        scratch_types=[
            pltpu.VMEM((_CH, _CHUNK), jnp.int32),
            pltpu.VMEM((_CH, _CHUNK), jnp.int32),
            pltpu.VMEM((_CHUNK, _D), jnp.float32),
            pltpu.VMEM_SHARED((_NP, _D), jnp.float32),
            pltpu.SemaphoreType.DMA,
        ],
    )
    def _sc_agg(x_hbm, src_hbm, dst_hbm, zeros_hbm, out_hbm,
                src_v, dst_v, rows_v, agg_sh, sem):
        c = lax.axis_index("c")
        s = lax.axis_index("s")
        pltpu.sync_copy(zeros_hbm.at[pl.ds(s * _RPT, _RPT)],
                        agg_sh.at[pl.ds(s * _RPT, _RPT)])
        pltpu.sync_copy(src_hbm.at[c, s], src_v)
        pltpu.sync_copy(dst_hbm.at[c, s], dst_v)
        plsc.subcore_barrier()

        def body(j, carry):
            pltpu.async_copy(x_hbm.at[src_v.at[j]], rows_v, sem).wait()
            pltpu.sync_copy(rows_v, agg_sh.at[dst_v.at[j]], add=True)
            return carry

        lax.fori_loop(0, _CH, body, 0)
        plsc.subcore_barrier()
        pltpu.sync_copy(agg_sh.at[pl.ds(s * _RPT, _RPT)],
                        out_hbm.at[c, pl.ds(s * _RPT, _RPT)])

    return _sc_agg


def _gin_mlp(x, agg, eps, wa, ba, wb, bb, g, be):
    h = (1.0 + eps) * x + agg
    h = jnp.maximum(_dot(h, wa) + ba, 0.0)
    h = jnp.maximum(_dot(h, wb) + bb, 0.0)
    return h * (g * _BNSCALE) + be


def _mlp_body(eps_ref, x_ref, agg_ref, wa_ref, ba_ref, wb_ref, bb_ref,
              g_ref, be_ref, out_ref):
    out_ref[...] = _gin_mlp(x_ref[...], agg_ref[0] + agg_ref[1], eps_ref[0],
                            wa_ref[...], ba_ref[...], wb_ref[...], bb_ref[...],
                            g_ref[...], be_ref[...])


_w_spec = pl.BlockSpec((_D, _D), lambda i: (0, 0))
_v_spec = pl.BlockSpec((1, _D), lambda i: (0, 0))

_mlp_call = pl.pallas_call(
    _mlp_body,
    grid=(_NBLK,),
    in_specs=[
        pl.BlockSpec(memory_space=pltpu.SMEM),
        pl.BlockSpec((_BN, _D), lambda i: (i, 0)),
        pl.BlockSpec((_NC, _BN, _D), lambda i: (0, i, 0)),
        _w_spec, _v_spec, _w_spec, _v_spec, _v_spec, _v_spec,
    ],
    out_specs=pl.BlockSpec((_BN, _D), lambda i: (i, 0)),
    out_shape=jax.ShapeDtypeStruct((_N, _D), jnp.float32),
)


def _final_body(eps_ref, x2_ref, agg_ref, wa_ref, ba_ref, wb_ref, bb_ref,
                g_ref, be_ref, x1_ref, batch_ref, emb_ref,
                w1_ref, b1_ref, w2_ref, b2_ref, w4_ref, b4_ref,
                out_ref, pooled_ref, counts_ref):
    i = pl.program_id(0)
    x2 = x2_ref[...]
    x3 = _gin_mlp(x2, agg_ref[0] + agg_ref[1], eps_ref[0],
                  wa_ref[...], ba_ref[...], wb_ref[...], bb_ref[...],
                  g_ref[...], be_ref[...])
    ids = batch_ref[0]
    gidx = lax.broadcasted_iota(jnp.int32, (_G, _BN), 0)
    oh = (gidx == ids).astype(jnp.float32)
    cat = jnp.concatenate([x1_ref[...], x2, x3], axis=1)
    contrib = _dot(oh, cat)
    cnt = jnp.sum(oh, axis=1, keepdims=True)

    @pl.when(i == 0)
    def _():
        pooled_ref[...] = jnp.zeros_like(pooled_ref)
        counts_ref[...] = jnp.zeros_like(counts_ref)

    pooled_ref[...] += contrib
    counts_ref[...] += jnp.broadcast_to(cnt, counts_ref.shape)

    @pl.when(i == _NBLK - 1)
    def _():
        c = jnp.maximum(counts_ref[...], 1.0)
        pooled = pooled_ref[...] / jnp.concatenate([c, c, c], axis=1)
        z = jnp.concatenate([pooled, emb_ref[...]], axis=1)
        z = jnp.maximum(_dot(z, w1_ref[...]) + b1_ref[...], 0.0)
        z = jnp.maximum(_dot(z, w2_ref[...]) + b2_ref[...], 0.0)
        z = _dot(z, w4_ref[...]) + b4_ref[...]
        m = jnp.max(z, axis=1, keepdims=True)
        lse = jnp.log(jnp.sum(jnp.exp(z - m), axis=1, keepdims=True)) + m
        out_ref[...] = z - lse


_final_call = pl.pallas_call(
    _final_body,
    grid=(_NBLK,),
    in_specs=[
        pl.BlockSpec(memory_space=pltpu.SMEM),
        pl.BlockSpec((_BN, _D), lambda i: (i, 0)),
        pl.BlockSpec((_NC, _BN, _D), lambda i: (0, i, 0)),
        _w_spec, _v_spec, _w_spec, _v_spec, _v_spec, _v_spec,
        pl.BlockSpec((_BN, _D), lambda i: (i, 0)),
        pl.BlockSpec((1, 1, _BN), lambda i: (i, 0, 0)),
        pl.BlockSpec((_G, _D), lambda i: (0, 0)),
        pl.BlockSpec((3 * _D + _D, 2 * _D), lambda i: (0, 0)),
        pl.BlockSpec((1, 2 * _D), lambda i: (0, 0)),
        pl.BlockSpec((2 * _D, _D), lambda i: (0, 0)),
        _v_spec,
        pl.BlockSpec((_D, _NCLS), lambda i: (0, 0)),
        pl.BlockSpec((1, _NCLS), lambda i: (0, 0)),
    ],
    out_specs=pl.BlockSpec((_G, _NCLS), lambda i: (0, 0)),
    out_shape=jax.ShapeDtypeStruct((_G, _NCLS), jnp.float32),
    scratch_shapes=[
        pltpu.VMEM((_G, 3 * _D), jnp.float32),
        pltpu.VMEM((_G, _D), jnp.float32),
    ],
)


def kernel(x, edge_index, batch, emb,
           eps0, W0a, b0a, W0b, b0b, g0, be0,
           eps1, W1a, b1a, W1b, b1b, g1, be1,
           eps2, W2a, b2a, W2b, b2b, g2, be2,
           W_lin1, b_lin1, W_lin2, b_lin2, W_lin4, b_lin4):
    src = edge_index[0]
    dst = edge_index[1]
    pad = _EPAD - _E
    src_p = jnp.concatenate([src, jnp.zeros((pad,), jnp.int32)])
    src_p = src_p.reshape(_NC, _NS, _CH, _CHUNK)
    dst_p = jnp.concatenate([dst, jnp.full((pad,), _N, jnp.int32)])
    dst_p = dst_p.reshape(_NC, _NS, _CH, _CHUNK)
    zeros = jnp.zeros((_NP, _D), jnp.float32)
    batch_r = batch.reshape(_NBLK, 1, _BN)

    def row(v):
        return v.reshape(1, -1)

    sc_agg = _get_sc_agg()
    agg1 = sc_agg(x, src_p, dst_p, zeros)
    x1 = _mlp_call(eps0.reshape(1), x, agg1, W0a, row(b0a), W0b, row(b0b),
                   row(g0), row(be0))
    agg2 = sc_agg(x1, src_p, dst_p, zeros)
    x2 = _mlp_call(eps1.reshape(1), x1, agg2, W1a, row(b1a), W1b, row(b1b),
                   row(g1), row(be1))
    agg3 = sc_agg(x2, src_p, dst_p, zeros)
    out = _final_call(eps2.reshape(1), x2, agg3, W2a, row(b2a), W2b, row(b2b),
                      row(g2), row(be2), x1, batch_r, emb,
                      W_lin1, row(b_lin1), W_lin2, row(b_lin2),
                      W_lin4, row(b_lin4))
    return out
